# baseline (speedup 1.0000x reference)
import functools

import jax
import jax.numpy as jnp
from jax import lax
from jax.experimental import pallas as pl
from jax.experimental.pallas import tpu as pltpu
from jax.experimental.pallas import tpu_sc as plsc

V = 1000000
D = 64
H = 256
B = 16384
EPS = 1e-5

@functools.cache
def _make_sc_gather():
    info = plsc.get_sparse_core_info()
    nc, ns = info.num_cores, info.num_subcores
    bpw = B // (nc * ns)

    def _gather_body(table_hbm, idx_hbm, out_hbm, idx_v, rows_v, sem):
        wid = lax.axis_index("s") * nc + lax.axis_index("c")
        base = wid * bpw
        pltpu.sync_copy(idx_hbm.at[pl.ds(base, bpw)], idx_v)
        pltpu.async_copy(table_hbm.at[idx_v], rows_v, sem).wait()
        pltpu.sync_copy(rows_v, out_hbm.at[pl.ds(base, bpw)])

    return pl.kernel(
        _gather_body,
        out_type=jax.ShapeDtypeStruct((B, D), jnp.float32),
        mesh=plsc.VectorSubcoreMesh(core_axis_name="c", subcore_axis_name="s"),
        scratch_types=[
            pltpu.VMEM((bpw,), jnp.int32),
            pltpu.VMEM((bpw, D), jnp.float32),
            pltpu.SemaphoreType.DMA,
        ],
        compiler_params=pltpu.CompilerParams(use_tc_tiling_on_sc=False),
    )


_BLK = 2048
_NB = B // _BLK


def _mlp_body(emb_ref, w1_ref, b1_ref, gamma_ref, beta_ref, w2_ref, b2_ref,
              out_ref):
    w1 = w1_ref[...]
    b1 = b1_ref[...]

    def stat_step(i, carry):
        s, ss = carry
        e = emb_ref[pl.ds(i * _BLK, _BLK), :]
        h = jnp.maximum(
            jnp.dot(e, w1, preferred_element_type=jnp.float32) + b1, 0.0)
        return (s + jnp.sum(h, axis=0, keepdims=True),
                ss + jnp.sum(h * h, axis=0, keepdims=True))

    zeros = jnp.zeros((1, H), jnp.float32)
    s, ss = lax.fori_loop(0, _NB, stat_step, (zeros, zeros))
    mean = s * (1.0 / B)
    var = ss * (1.0 / B) - mean * mean
    scale = gamma_ref[...] * lax.rsqrt(var + EPS)
    w2p = w2_ref[...] * scale.reshape(H, 1)
    bias = jnp.dot(beta_ref[...] - mean * scale, w2_ref[...],
                   preferred_element_type=jnp.float32) + b2_ref[...]

    def out_step(i, _):
        e = emb_ref[pl.ds(i * _BLK, _BLK), :]
        h = jnp.maximum(
            jnp.dot(e, w1, preferred_element_type=jnp.float32) + b1, 0.0)
        out_ref[pl.ds(i * _BLK, _BLK), :] = jnp.maximum(
            jnp.dot(h, w2p, preferred_element_type=jnp.float32) + bias, 0.0)
        return 0

    lax.fori_loop(0, _NB, out_step, 0)


def _mlp(emb, W1, b1, gamma, beta, W2, b2):
    return pl.pallas_call(
        _mlp_body,
        out_shape=jax.ShapeDtypeStruct((B, H), jnp.float32),
    )(emb, W1, b1.reshape(1, H), gamma.reshape(1, H), beta.reshape(1, H),
      W2, b2.reshape(1, H))


def kernel(target_ids, table, W1, b1, gamma, beta, W2, b2):
    emb = _make_sc_gather()(table, target_ids.astype(jnp.int32))
    return _mlp(emb, W1, b1, gamma, beta, W2, b2)


# speedup vs baseline: 1.7165x; 1.7165x over previous
import functools

import jax
import jax.numpy as jnp
from jax import lax
from jax.experimental import pallas as pl
from jax.experimental.pallas import tpu as pltpu
from jax.experimental.pallas import tpu_sc as plsc

V = 1000000
D = 64
H = 256
B = 16384
EPS = 1e-5

@functools.cache
def _make_sc_gather():
    info = plsc.get_sparse_core_info()
    nc, ns = info.num_cores, info.num_subcores
    bpw = B // (nc * ns)

    def _gather_body(table_hbm, idx_hbm, out_hbm, idx_v, rows_v, sem):
        wid = lax.axis_index("s") * nc + lax.axis_index("c")
        base = wid * bpw
        pltpu.sync_copy(idx_hbm.at[pl.ds(base, bpw)], idx_v)

        def issue(g, carry):
            vec = idx_v[pl.ds(g * 16, 16)]
            for l in range(16):
                pltpu.async_copy(
                    table_hbm.at[pl.ds(vec[l], 1)],
                    rows_v.at[pl.ds(g * 16 + l, 1)], sem)
            return carry

        lax.fori_loop(0, bpw // 16, issue, 0)
        pltpu.make_async_copy(
            table_hbm.at[pl.ds(0, bpw)], rows_v, sem).wait()
        pltpu.sync_copy(rows_v, out_hbm.at[pl.ds(base, bpw)])

    return pl.kernel(
        _gather_body,
        out_type=jax.ShapeDtypeStruct((B, D), jnp.float32),
        mesh=plsc.VectorSubcoreMesh(core_axis_name="c", subcore_axis_name="s"),
        scratch_types=[
            pltpu.VMEM((bpw,), jnp.int32),
            pltpu.VMEM((bpw, D), jnp.float32),
            pltpu.SemaphoreType.DMA,
        ],
    )


_BLK = 2048
_NB = B // _BLK


def _mlp_body(emb_ref, w1_ref, b1_ref, gamma_ref, beta_ref, w2_ref, b2_ref,
              out_ref):
    w1 = w1_ref[...]
    b1 = b1_ref[...]

    def stat_step(i, carry):
        s, ss = carry
        e = emb_ref[pl.ds(i * _BLK, _BLK), :]
        h = jnp.maximum(
            jnp.dot(e, w1, preferred_element_type=jnp.float32) + b1, 0.0)
        return (s + jnp.sum(h, axis=0, keepdims=True),
                ss + jnp.sum(h * h, axis=0, keepdims=True))

    zeros = jnp.zeros((1, H), jnp.float32)
    s, ss = lax.fori_loop(0, _NB, stat_step, (zeros, zeros))
    mean = s * (1.0 / B)
    var = ss * (1.0 / B) - mean * mean
    scale = gamma_ref[...] * lax.rsqrt(var + EPS)
    w2p = w2_ref[...] * scale.reshape(H, 1)
    bias = jnp.dot(beta_ref[...] - mean * scale, w2_ref[...],
                   preferred_element_type=jnp.float32) + b2_ref[...]

    def out_step(i, _):
        e = emb_ref[pl.ds(i * _BLK, _BLK), :]
        h = jnp.maximum(
            jnp.dot(e, w1, preferred_element_type=jnp.float32) + b1, 0.0)
        out_ref[pl.ds(i * _BLK, _BLK), :] = jnp.maximum(
            jnp.dot(h, w2p, preferred_element_type=jnp.float32) + bias, 0.0)
        return 0

    lax.fori_loop(0, _NB, out_step, 0)


def _mlp(emb, W1, b1, gamma, beta, W2, b2):
    return pl.pallas_call(
        _mlp_body,
        out_shape=jax.ShapeDtypeStruct((B, H), jnp.float32),
    )(emb, W1, b1.reshape(1, H), gamma.reshape(1, H), beta.reshape(1, H),
      W2, b2.reshape(1, H))


def kernel(target_ids, table, W1, b1, gamma, beta, W2, b2):
    emb = _make_sc_gather()(table, target_ids.astype(jnp.int32))
    return _mlp(emb, W1, b1, gamma, beta, W2, b2)
